# Initial kernel scaffold; baseline (speedup 1.0000x reference)
import jax
import jax.numpy as jnp
from jax.experimental import pallas as pl


def kernel(embeddings, edge_index):
    raise NotImplementedError("write your pallas kernel here")


# speedup vs baseline: 50.3132x; 50.3132x over previous
import functools

import jax
import jax.numpy as jnp
from jax import lax
from jax.experimental import pallas as pl
from jax.experimental.pallas import tpu as pltpu
from jax.experimental.pallas import tpu_sc as plsc

N_NODES = 10000
N_EDGES = 320000
DIM = 128
LAMBDA_WEIGHT = 0.1
L = 16
NW = 32
CHUNK = 2 * N_EDGES // NW

_mesh = plsc.VectorSubcoreMesh(core_axis_name="c", subcore_axis_name="s")


@functools.partial(
    pl.kernel,
    out_type=jax.ShapeDtypeStruct((NW, N_NODES), jnp.int32),
    mesh=_mesh,
    scratch_types=[
        pltpu.VMEM((CHUNK,), jnp.int32),
        pltpu.VMEM((N_NODES,), jnp.int32),
    ],
    compiler_params=pltpu.CompilerParams(needs_layout_passes=False),
)
def _hist_kernel(idx_hbm, out_hbm, idx_v, hist_v):
    c = lax.axis_index("c")
    s = lax.axis_index("s")
    wid = s * 2 + c
    base = wid * CHUNK
    pltpu.sync_copy(idx_hbm.at[pl.ds(base, CHUNK)], idx_v)

    zeros = jnp.zeros((L,), jnp.int32)

    def zero_body(i, carry):
        hist_v[pl.ds(i * L, L)] = zeros
        return carry

    lax.fori_loop(0, N_NODES // L, zero_body, 0, unroll=8)

    ones = jnp.ones((L,), jnp.int32)

    def body(i, carry):
        idx = idx_v[pl.ds(i * L, L)]
        plsc.addupdate_scatter(hist_v, [idx], ones)
        return carry

    lax.fori_loop(0, CHUNK // L, body, 0, unroll=8)

    pltpu.sync_copy(hist_v, out_hbm.at[wid])


def _reduce_body(emb_ref, parts_ref, out_ref):
    parts = parts_ref[...].astype(jnp.float32)
    cnt_src = jnp.sum(parts[: NW // 2], axis=0, keepdims=True)
    cnt_dst = jnp.sum(parts[NW // 2 :], axis=0, keepdims=True)
    emb = emb_ref[...]
    dot = functools.partial(
        lax.dot_general,
        precision=lax.Precision.HIGHEST,
        preferred_element_type=jnp.float32,
    )
    S = dot(cnt_dst, emb, (((1,), (0,)), ((), ())))
    T = dot(cnt_src, emb, (((1,), (0,)), ((), ())))
    ssq = jnp.sum(emb * emb, axis=1, keepdims=True)
    R = dot(cnt_src, ssq, (((1,), (0,)), ((), ())))
    TS = jnp.sum(T * S)
    SS = jnp.sum(S * S)
    loss = (LAMBDA_WEIGHT / (N_EDGES * DIM)) * (
        R[0, 0] - 2.0 * TS + N_EDGES * SS
    )
    out_ref[0, 0] = loss


def kernel(embeddings, edge_index):
    idx = edge_index.reshape(-1).astype(jnp.int32)
    parts = _hist_kernel(idx)
    loss = pl.pallas_call(
        _reduce_body,
        out_shape=jax.ShapeDtypeStruct((1, 1), jnp.float32),
        out_specs=pl.BlockSpec(memory_space=pltpu.SMEM),
    )(embeddings, parts)
    return loss[0, 0]
